# Initial kernel scaffold; baseline (speedup 1.0000x reference)
import jax
import jax.numpy as jnp
from jax.experimental import pallas as pl


def kernel(input_ids, token_type_ids, word_emb, pos_emb, tok_emb):
    raise NotImplementedError("write your pallas kernel here")


# speedup vs baseline: 7.6221x; 7.6221x over previous
import functools

import jax
import jax.numpy as jnp
from jax import lax
from jax.experimental import pallas as pl
from jax.experimental.pallas import tpu as pltpu
from jax.experimental.pallas import tpu_sc as plsc

VOCAB = 100000
HID = 128
CTX = 200
NROW = 1024 * 200
NC = 2
NS = 16
NW = NC * NS
RPW = NROW // NW
CHUNK = 128
NCHUNK = RPW // CHUNK
LANES = 16


def _combine_body(pos_ref, tok_ref, out_ref):
    out_ref[0:CTX, :] = pos_ref[...] + tok_ref[0:1, :]
    out_ref[CTX:2 * CTX, :] = pos_ref[...] + tok_ref[1:2, :]


def _build_combined(pos_emb, tok_emb):
    return pl.pallas_call(
        _combine_body,
        out_shape=jax.ShapeDtypeStruct((2 * CTX, HID), jnp.float32),
    )(pos_emb, tok_emb)


_sc_mesh = plsc.VectorSubcoreMesh(core_axis_name="c", subcore_axis_name="s")


@functools.partial(
    pl.kernel,
    out_type=jax.ShapeDtypeStruct((NROW, HID), jnp.float32),
    mesh=_sc_mesh,
    scratch_types=[
        pltpu.VMEM((CHUNK,), jnp.int32),
        pltpu.VMEM((CHUNK,), jnp.int32),
        pltpu.VMEM((CHUNK,), jnp.int32),
        pltpu.VMEM((CHUNK, HID), jnp.float32),
        pltpu.VMEM((CHUNK, HID), jnp.float32),
        pltpu.SemaphoreType.DMA,
    ],
)
def _sc_embed(word_hbm, comb_hbm, ids_hbm, tt_hbm, out_hbm,
              widx, ttv, cidx, wrows, crows, sem):
    wid = lax.axis_index("s") * NC + lax.axis_index("c")
    row0 = wid * RPW

    def chunk_body(ci, _):
        base = row0 + ci * CHUNK
        pltpu.sync_copy(ids_hbm.at[pl.ds(base, CHUNK)], widx)
        pltpu.sync_copy(tt_hbm.at[pl.ds(base, CHUNK)], ttv)

        def idx_body(j, _):
            o = j * LANES
            n = base + o + lax.iota(jnp.int32, LANES)
            s = n % CTX
            cidx[pl.ds(o, LANES)] = ttv[pl.ds(o, LANES)] * CTX + s
            return 0

        lax.fori_loop(0, CHUNK // LANES, idx_body, 0)

        g1 = pltpu.async_copy(word_hbm.at[widx], wrows, sem)
        g2 = pltpu.async_copy(comb_hbm.at[cidx], crows, sem)
        g1.wait()
        g2.wait()

        def add_body(r, _):
            for j in range(HID // LANES):
                sl = pl.ds(j * LANES, LANES)
                wrows[r, sl] = wrows[r, sl] + crows[r, sl]
            return 0

        lax.fori_loop(0, CHUNK, add_body, 0)
        pltpu.sync_copy(wrows, out_hbm.at[pl.ds(base, CHUNK)])
        return 0

    lax.fori_loop(0, NCHUNK, chunk_body, 0)


def kernel(input_ids, token_type_ids, word_emb, pos_emb, tok_emb):
    combined = _build_combined(pos_emb, tok_emb)
    ids_flat = input_ids.reshape(-1)
    tt_flat = token_type_ids.reshape(-1)
    out = _sc_embed(word_emb, combined, ids_flat, tt_flat)
    return out.reshape(input_ids.shape[0], input_ids.shape[1], HID)


# speedup vs baseline: 10.7654x; 1.4124x over previous
import functools

import jax
import jax.numpy as jnp
from jax import lax
from jax.experimental import pallas as pl
from jax.experimental.pallas import tpu as pltpu
from jax.experimental.pallas import tpu_sc as plsc

VOCAB = 100000
HID = 128
CTX = 200
NROW = 1024 * 200
NC = 2
NS = 16
NW = NC * NS
RPW = NROW // NW
CHUNK = 128
NCHUNK = RPW // CHUNK
LANES = 16


def _combine_body(pos_ref, tok_ref, out_ref):
    out_ref[0:CTX, :] = pos_ref[...] + tok_ref[0:1, :]
    out_ref[CTX:2 * CTX, :] = pos_ref[...] + tok_ref[1:2, :]


def _build_combined(pos_emb, tok_emb):
    return pl.pallas_call(
        _combine_body,
        out_shape=jax.ShapeDtypeStruct((2 * CTX, HID), jnp.float32),
    )(pos_emb, tok_emb)


_sc_mesh = plsc.VectorSubcoreMesh(core_axis_name="c", subcore_axis_name="s")


@functools.partial(
    pl.kernel,
    out_type=jax.ShapeDtypeStruct((NROW, HID), jnp.float32),
    mesh=_sc_mesh,
    scratch_types=[
        pltpu.VMEM((2, CHUNK), jnp.int32),
        pltpu.VMEM((2, CHUNK), jnp.int32),
        pltpu.VMEM((2, CHUNK), jnp.int32),
        pltpu.VMEM((2, CHUNK, HID), jnp.float32),
        pltpu.VMEM((2, CHUNK, HID), jnp.float32),
        pltpu.SemaphoreType.DMA,
        pltpu.SemaphoreType.DMA,
    ],
)
def _sc_embed(word_hbm, comb_hbm, ids_hbm, tt_hbm, out_hbm,
              widx, ttv, cidx, wrows, crows, gsem0, gsem1):
    wid = lax.axis_index("s") * NC + lax.axis_index("c")
    row0 = wid * RPW
    gsems = (gsem0, gsem1)

    def prep_idx(ci, b):
        base = row0 + ci * CHUNK
        pltpu.sync_copy(ids_hbm.at[pl.ds(base, CHUNK)], widx.at[b])
        pltpu.sync_copy(tt_hbm.at[pl.ds(base, CHUNK)], ttv.at[b])

        def idx_body(j, _):
            o = j * LANES
            n = base + o + lax.iota(jnp.int32, LANES)
            s = n % CTX
            cidx[b, pl.ds(o, LANES)] = ttv[b, pl.ds(o, LANES)] * CTX + s
            return 0

        lax.fori_loop(0, CHUNK // LANES, idx_body, 0)

    def start_gather(b):
        pltpu.async_copy(word_hbm.at[widx.at[b]], wrows.at[b], gsems[b])
        pltpu.async_copy(comb_hbm.at[cidx.at[b]], crows.at[b], gsems[b])

    def wait_gather(b):
        pltpu.make_async_copy(word_hbm.at[widx.at[b]], wrows.at[b],
                              gsems[b]).wait()
        pltpu.make_async_copy(comb_hbm.at[cidx.at[b]], crows.at[b],
                              gsems[b]).wait()

    def finish_chunk(ci, b):
        wait_gather(b)

        def add_body(r, _):
            for j in range(HID // LANES):
                sl = pl.ds(j * LANES, LANES)
                wrows[b, r, sl] = wrows[b, r, sl] + crows[b, r, sl]
            return 0

        lax.fori_loop(0, CHUNK, add_body, 0)
        base = row0 + ci * CHUNK
        pltpu.sync_copy(wrows.at[b], out_hbm.at[pl.ds(base, CHUNK)])

    prep_idx(0, 0)
    start_gather(0)
    prep_idx(1, 1)
    start_gather(1)

    def outer(oi, _):
        for b in range(2):
            ci = oi * 2 + b
            finish_chunk(ci, b)

            @pl.when(ci + 2 < NCHUNK)
            def _():
                prep_idx(ci + 2, b)
                start_gather(b)
        return 0

    lax.fori_loop(0, NCHUNK // 2, outer, 0)


def kernel(input_ids, token_type_ids, word_emb, pos_emb, tok_emb):
    combined = _build_combined(pos_emb, tok_emb)
    ids_flat = input_ids.reshape(-1)
    tt_flat = token_type_ids.reshape(-1)
    out = _sc_embed(word_emb, combined, ids_flat, tt_flat)
    return out.reshape(input_ids.shape[0], input_ids.shape[1], HID)
